# Initial kernel scaffold; baseline (speedup 1.0000x reference)
import jax
import jax.numpy as jnp
from jax.experimental import pallas as pl


def kernel(x, main_w1, main_bn1_gamma, main_bn1_beta, main_bn1_mean, main_bn1_var, main_dw, main_bn2_gamma, main_bn2_beta, main_bn2_mean, main_bn2_var, main_w3, main_bn3_gamma, main_bn3_beta, main_bn3_mean, main_bn3_var):
    raise NotImplementedError("write your pallas kernel here")


# speedup vs baseline: 1.1637x; 1.1637x over previous
import functools

import jax
import jax.numpy as jnp
from jax.experimental import pallas as pl
from jax.experimental.pallas import tpu as pltpu


_VMEM_LIMIT = 64 * 1024 * 1024


def _shift_lanes(a, d):
    if d == 0:
        return a
    C = a.shape[0]
    if d > 0:
        return jnp.concatenate(
            [a[:, d:], jnp.zeros((C, d), a.dtype)], axis=1)
    return jnp.concatenate(
        [jnp.zeros((C, -d), a.dtype), a[:, :a.shape[1] + d]], axis=1)


def _block_kernel(x_ref, sel_ref, w1_ref, b1_ref, dw_ref, w3_ref, b3_ref,
                  o_ref, *, ksize, pad, W, Bb):
    HW = x_ref.shape[-1]
    sel = sel_ref[...]
    w1 = w1_ref[...]
    b1 = b1_ref[...]
    dw = dw_ref[...]
    w3 = w3_ref[...]
    b3 = b3_ref[...]

    xpos = jax.lax.broadcasted_iota(jnp.int32, (1, HW), 1) % W
    premask = {0: None}
    for dx in range(ksize):
        ox = dx - pad
        if ox == 0:
            continue
        m = (xpos >= max(0, ox)) & (xpos < W + min(0, ox))
        premask[ox] = m.astype(jnp.float32)

    for b in range(Bb):
        xb = x_ref[b].astype(jnp.bfloat16)
        o_ref[b, 0] = jnp.dot(sel, xb, preferred_element_type=jnp.float32)
        h = jnp.maximum(
            jnp.dot(w1, xb, preferred_element_type=jnp.float32) + b1, 0.0)
        hm = {ox: (h if m is None else h * m) for ox, m in premask.items()}
        acc = jnp.zeros_like(h)
        for dy in range(ksize):
            oy = dy - pad
            for dx in range(ksize):
                ox = dx - pad
                t = dy * ksize + dx
                g = _shift_lanes(hm[ox], oy * W + ox)
                acc = acc + g * dw[:, t:t + 1]
        y = jnp.maximum(
            jnp.dot(w3, acc.astype(jnp.bfloat16),
                    preferred_element_type=jnp.float32) + b3, 0.0)
        o_ref[b, 1] = y


def _const_spec(a):
    zeros = (0,) * a.ndim
    return pl.BlockSpec(a.shape, lambda b: zeros)


def _pick_block_batch(B, target_steps=8):
    cap = max(1, B // target_steps)
    for bb in range(cap, 0, -1):
        if B % bb == 0:
            return bb
    return 1


def _fold_bn(gamma, beta, mean, var, eps=1e-5):
    scale = gamma / jnp.sqrt(var + eps)
    bias = beta - mean * scale
    return scale, bias


def kernel(x, main_w1, main_bn1_gamma, main_bn1_beta, main_bn1_mean,
           main_bn1_var, main_dw, main_bn2_gamma, main_bn2_beta, main_bn2_mean,
           main_bn2_var, main_w3, main_bn3_gamma, main_bn3_beta, main_bn3_mean,
           main_bn3_var):
    B, C, H, W = x.shape
    inp = C // 2
    mid = main_w1.shape[0]
    HW = H * W
    ksize = main_dw.shape[-1]
    pad = ksize // 2
    outputs = main_w3.shape[0]
    assert outputs == inp

    s1, b1 = _fold_bn(main_bn1_gamma, main_bn1_beta, main_bn1_mean, main_bn1_var)
    s2, b2 = _fold_bn(main_bn2_gamma, main_bn2_beta, main_bn2_mean, main_bn2_var)
    s3, b3 = _fold_bn(main_bn3_gamma, main_bn3_beta, main_bn3_mean, main_bn3_var)
    w1 = main_w1[:, :, 0, 0] * s1[:, None]
    dw = main_dw[:, 0].reshape(mid, -1) * s2[:, None]
    w3 = main_w3[:, :, 0, 0] * s3[:, None]
    b3f = b3 + s3 * (main_w3[:, :, 0, 0] @ b2)

    w1e = jnp.zeros((mid, C), jnp.float32).at[:, 1::2].set(w1)
    sel = jnp.zeros((inp, C), jnp.float32).at[jnp.arange(inp), 2 * jnp.arange(inp)].set(1.0)

    selb = sel.astype(jnp.bfloat16)
    w1b = w1e.astype(jnp.bfloat16)
    w3b = w3.astype(jnp.bfloat16)
    b1c = b1[:, None]
    b3c = b3f[:, None]

    Bb = _pick_block_batch(B)
    x3 = x.reshape(B, C, HW)

    params = (selb, w1b, b1c, dw, w3b, b3c)
    kern = functools.partial(_block_kernel, ksize=ksize, pad=pad, W=W, Bb=Bb)
    out = pl.pallas_call(
        kern,
        out_shape=jax.ShapeDtypeStruct((B, 2, inp, HW), jnp.float32),
        grid=(B // Bb,),
        in_specs=[pl.BlockSpec((Bb, C, HW), lambda b: (b, 0, 0))]
                 + [_const_spec(a) for a in params],
        out_specs=pl.BlockSpec((Bb, 2, inp, HW), lambda b: (b, 0, 0, 0)),
        compiler_params=pltpu.CompilerParams(
            dimension_semantics=("parallel",),
            vmem_limit_bytes=_VMEM_LIMIT),
    )(x3, *params)
    return out.reshape(B, 2 * inp, H, W)
